# baseline (speedup 1.0000x reference)
import functools

import jax
import jax.numpy as jnp
import numpy as np
from jax import lax
from jax.experimental import pallas as pl
from jax.experimental.pallas import tpu as pltpu
from jax.experimental.pallas import tpu_sc as plsc

N_LEVELS = 16
N_CHANNELS = 2
LOG2_TABLE = 19
N_VOLUMES = 64
LOCAL_SIZE = 1 << LOG2_TABLE
POOL_SIZE = LOCAL_SIZE * N_LEVELS
RES_BASE_POW2 = 3.0
RES_FINE_POW2 = 10.0
N_POINTS = 131072

_rng = np.random.RandomState(123)
_PRIMES_NP = (
    _rng.randint(1 << 20, 1 << 30, size=(N_LEVELS, 3)).astype(np.uint32)
    * np.uint32(2)
    + np.uint32(1)
).astype(np.int64).astype(np.int32)

_NC = 2
_NS = 16
_NW = _NC * _NS
_CHUNK = N_POINTS // _NW
_G = 16
_NGROUPS = _CHUNK // _G
_MASK = LOCAL_SIZE - 1
_OUT_W = N_LEVELS * N_CHANNELS


def _body(points_hbm, anchors_hbm, feat_hbm, bias_hbm, res_hbm,
          out_hbm,
          pts_v, anch_v, bias_v, res_v,
          idx_buf, w_buf, feat_buf, out_buf, sem, outsem):
    wid = lax.axis_index("s") * _NC + lax.axis_index("c")
    base = wid * _CHUNK

    pltpu.sync_copy(points_hbm.at[pl.ds(base * 3, _CHUNK * 3)], pts_v)
    pltpu.sync_copy(anchors_hbm.at[pl.ds(base, _CHUNK)], anch_v)
    pltpu.sync_copy(bias_hbm, bias_v)
    pltpu.sync_copy(res_hbm, res_v)

    p_iota = lax.iota(jnp.int32, 16)
    p3 = p_iota * 3
    p2 = p_iota * 2
    pout = p_iota * _OUT_W
    zero_i = jnp.zeros((16,), jnp.int32)

    def group_body(g, carry):
        prow = g * (_G * 3) + p3
        x = plsc.load_gather(pts_v, [prow])
        y = plsc.load_gather(pts_v, [prow + 1])
        z = plsc.load_gather(pts_v, [prow + 2])
        anch = anch_v[pl.ds(g * _G, _G)]
        brow = anch * (N_LEVELS * 3)

        for l in range(N_LEVELS):
            l3 = l * 3
            res = plsc.load_gather(res_v, [zero_i + (l + 1)])
            px = int(_PRIMES_NP[l, 0])
            py = int(_PRIMES_NP[l, 1])
            pz = int(_PRIMES_NP[l, 2])
            bx = plsc.load_gather(bias_v, [brow + l3])
            by = plsc.load_gather(bias_v, [brow + (l3 + 1)])
            bz = plsc.load_gather(bias_v, [brow + (l3 + 2)])
            ptx = x * res + bx
            pty = y * res + by
            ptz = z * res + bz
            ix = ptx.astype(jnp.int32)
            iy = pty.astype(jnp.int32)
            iz = ptz.astype(jnp.int32)
            fx = ptx - ix.astype(jnp.float32)
            fy = pty - iy.astype(jnp.float32)
            fz = ptz - iz.astype(jnp.float32)
            hx0 = ix * px
            hx1 = hx0 + px
            hy0 = iy * py
            hy1 = hy0 + py
            hz0 = iz * pz
            hz1 = hz0 + pz
            hx = (hx0, hx1)
            hy = (hy0, hy1)
            hz = (hz0, hz1)
            wx = (1.0 - fx, fx)
            wy = (1.0 - fy, fy)
            wz = (1.0 - fz, fz)
            lvl_off2 = l * (2 * LOCAL_SIZE)
            ibase = l * 128
            ibase2 = l * 256
            for k in range(8):
                bxk, byk, bzk = (k >> 2) & 1, (k >> 1) & 1, k & 1
                hm = (hx[bxk] ^ hy[byk] ^ hz[bzk]) & _MASK
                h2 = hm + hm + lvl_off2
                posv = ibase2 + k * 32 + p2
                plsc.store_scatter(idx_buf, [posv], h2)
                plsc.store_scatter(idx_buf, [posv + 1], h2 + 1)
                w_buf[pl.ds(ibase + k * 16, 16)] = wx[bxk] * wy[byk] * wz[bzk]
            for j in range(2):
                sl = pl.ds(ibase2 + j * 128, 128)
                pltpu.async_copy(feat_hbm.at[idx_buf.at[sl]],
                                 feat_buf.at[sl], sem)

        for l in range(N_LEVELS):
            for j in range(2):
                sl = pl.ds(l * 256 + j * 128, 128)
                pltpu.make_async_copy(feat_hbm.at[idx_buf.at[sl]],
                                      feat_buf.at[sl], sem).wait()

        for l in range(N_LEVELS):
            ibase = l * 128
            fbase = l * 256
            acc0 = jnp.zeros((16,), jnp.float32)
            acc1 = jnp.zeros((16,), jnp.float32)
            for k in range(8):
                fpos = fbase + k * 32 + p2
                f0 = plsc.load_gather(feat_buf, [fpos])
                f1 = plsc.load_gather(feat_buf, [fpos + 1])
                w = w_buf[pl.ds(ibase + k * 16, 16)]
                acc0 = acc0 + w * f0
                acc1 = acc1 + w * f1
            plsc.store_scatter(out_buf, [pout + 2 * l], acc0)
            plsc.store_scatter(out_buf, [pout + (2 * l + 1)], acc1)

        pltpu.async_copy(
            out_buf,
            out_hbm.at[pl.ds((base + g * _G) * _OUT_W, _G * _OUT_W)],
            outsem).wait()
        return carry

    lax.fori_loop(0, _NGROUPS, group_body, 0)


@jax.jit
def _run(points, anchors, feat_pool, bias_pool, res):
    mesh = plsc.VectorSubcoreMesh(core_axis_name="c", subcore_axis_name="s")
    f = functools.partial(
        pl.kernel,
        out_type=jax.ShapeDtypeStruct((N_POINTS * _OUT_W,), jnp.float32),
        mesh=mesh,
        scratch_types=[
            pltpu.VMEM((_CHUNK * 3,), jnp.float32),
            pltpu.VMEM((_CHUNK,), jnp.int32),
            pltpu.VMEM((N_VOLUMES * N_LEVELS * 3,), jnp.float32),
            pltpu.VMEM((24,), jnp.float32),
            pltpu.VMEM((N_LEVELS * 256,), jnp.int32),
            pltpu.VMEM((N_LEVELS * 128,), jnp.float32),
            pltpu.VMEM((N_LEVELS * 256,), jnp.float32),
            pltpu.VMEM((_G * _OUT_W,), jnp.float32),
            pltpu.SemaphoreType.DMA,
            pltpu.SemaphoreType.DMA,
        ],
        compiler_params=pltpu.CompilerParams(needs_layout_passes=False),
    )(_body)
    return f(points, anchors, feat_pool, bias_pool, res)


def kernel(points, anchors, feat_pool, bias_pool):
    levels = jnp.arange(N_LEVELS, dtype=jnp.float32)
    res = jnp.exp2(RES_BASE_POW2
                   + (RES_FINE_POW2 - RES_BASE_POW2) * levels / (N_LEVELS - 1))
    res_pad = jnp.zeros((24,), jnp.float32).at[1:N_LEVELS + 1].set(res)
    out = _run(points.reshape(-1), anchors, feat_pool.reshape(-1),
               bias_pool.reshape(-1), res_pad)
    return out.reshape(N_POINTS, _OUT_W)


# speedup vs baseline: 1.0062x; 1.0062x over previous
import functools

import jax
import jax.numpy as jnp
import numpy as np
from jax import lax
from jax.experimental import pallas as pl
from jax.experimental.pallas import tpu as pltpu
from jax.experimental.pallas import tpu_sc as plsc

N_LEVELS = 16
N_CHANNELS = 2
LOG2_TABLE = 19
N_VOLUMES = 64
LOCAL_SIZE = 1 << LOG2_TABLE
POOL_SIZE = LOCAL_SIZE * N_LEVELS
RES_BASE_POW2 = 3.0
RES_FINE_POW2 = 10.0
N_POINTS = 131072

_rng = np.random.RandomState(123)
_PRIMES_NP = (
    _rng.randint(1 << 20, 1 << 30, size=(N_LEVELS, 3)).astype(np.uint32)
    * np.uint32(2)
    + np.uint32(1)
).astype(np.int64).astype(np.int32)

_NC = 2
_NS = 16
_NW = _NC * _NS
_CHUNK = N_POINTS // _NW
_G = 16
_NGROUPS = _CHUNK // _G
_MASK = LOCAL_SIZE - 1
_OUT_W = N_LEVELS * N_CHANNELS

_NPAR_I = 7
_PARAMS_I_NP = np.zeros((N_LEVELS, _NPAR_I, 16), np.int32)
for _l in range(N_LEVELS):
    _PARAMS_I_NP[_l, 0, :] = _PRIMES_NP[_l, 0]
    _PARAMS_I_NP[_l, 1, :] = _PRIMES_NP[_l, 1]
    _PARAMS_I_NP[_l, 2, :] = _PRIMES_NP[_l, 2]
    _PARAMS_I_NP[_l, 3, :] = _l * (2 * LOCAL_SIZE)
    _PARAMS_I_NP[_l, 4, :] = 3 * _l
    _PARAMS_I_NP[_l, 5, :] = 2 * _l
    _PARAMS_I_NP[_l, 6, :] = 256 * _l
_PARAMS_I_NP = _PARAMS_I_NP.reshape(-1)


def _body(points_hbm, anchors_hbm, feat_hbm, bias_hbm, res_hbm, pari_hbm,
          out_hbm,
          pts_v, anch_v, bias_v, res_v, pari_v,
          idx_buf, w_buf, feat_buf, out_buf, sem, outsem):
    wid = lax.axis_index("s") * _NC + lax.axis_index("c")
    base = wid * _CHUNK

    pltpu.sync_copy(points_hbm.at[pl.ds(base * 3, _CHUNK * 3)], pts_v)
    pltpu.sync_copy(anchors_hbm.at[pl.ds(base, _CHUNK)], anch_v)
    pltpu.sync_copy(bias_hbm, bias_v)
    pltpu.sync_copy(res_hbm, res_v)
    pltpu.sync_copy(pari_hbm, pari_v)

    p_iota = lax.iota(jnp.int32, 16)
    p3 = p_iota * 3
    p2 = p_iota * 2
    pout = p_iota * _OUT_W
    zero_i = jnp.zeros((16,), jnp.int32)

    def group_body(g, carry):
        prow = g * (_G * 3) + p3
        x = plsc.load_gather(pts_v, [prow])
        y = plsc.load_gather(pts_v, [prow + 1])
        z = plsc.load_gather(pts_v, [prow + 2])
        anch = anch_v[pl.ds(g * _G, _G)]
        brow = anch * (N_LEVELS * 3)

        def level_body(l, c):
            pb = l * (_NPAR_I * 16)
            pxv = pari_v[pl.ds(pb, 16)]
            pyv = pari_v[pl.ds(pb + 16, 16)]
            pzv = pari_v[pl.ds(pb + 32, 16)]
            loff2 = pari_v[pl.ds(pb + 48, 16)]
            l3v = pari_v[pl.ds(pb + 64, 16)]
            l256 = pari_v[pl.ds(pb + 96, 16)]
            res = res_v[pl.ds(l * 16, 16)]
            bx = plsc.load_gather(bias_v, [brow + l3v])
            by = plsc.load_gather(bias_v, [brow + (l3v + 1)])
            bz = plsc.load_gather(bias_v, [brow + (l3v + 2)])
            ptx = x * res + bx
            pty = y * res + by
            ptz = z * res + bz
            ix = ptx.astype(jnp.int32)
            iy = pty.astype(jnp.int32)
            iz = ptz.astype(jnp.int32)
            fx = ptx - ix.astype(jnp.float32)
            fy = pty - iy.astype(jnp.float32)
            fz = ptz - iz.astype(jnp.float32)
            hx0 = ix * pxv
            hx1 = hx0 + pxv
            hy0 = iy * pyv
            hy1 = hy0 + pyv
            hz0 = iz * pzv
            hz1 = hz0 + pzv
            hx = (hx0, hx1)
            hy = (hy0, hy1)
            hz = (hz0, hz1)
            wx = (1.0 - fx, fx)
            wy = (1.0 - fy, fy)
            wz = (1.0 - fz, fz)
            ibase = l * 128
            ibase2 = l * 256
            for k in range(8):
                bxk, byk, bzk = (k >> 2) & 1, (k >> 1) & 1, k & 1
                hm = (hx[bxk] ^ hy[byk] ^ hz[bzk]) & _MASK
                h2 = hm + hm + loff2
                posv = l256 + (k * 32) + p2
                plsc.store_scatter(idx_buf, [posv], h2)
                plsc.store_scatter(idx_buf, [posv + 1], h2 + 1)
                w_buf[pl.ds(ibase + k * 16, 16)] = wx[bxk] * wy[byk] * wz[bzk]
            for j in range(2):
                sl = pl.ds(ibase2 + j * 128, 128)
                pltpu.async_copy(feat_hbm.at[idx_buf.at[sl]],
                                 feat_buf.at[sl], sem)
            return c

        lax.fori_loop(0, N_LEVELS, level_body, 0)

        def drain_body(l, c):
            for j in range(2):
                sl = pl.ds(l * 256 + j * 128, 128)
                pltpu.make_async_copy(feat_hbm.at[idx_buf.at[sl]],
                                      feat_buf.at[sl], sem).wait()
            return c

        lax.fori_loop(0, N_LEVELS, drain_body, 0)

        def acc_body(l, c):
            pb = l * (_NPAR_I * 16)
            l2v = pari_v[pl.ds(pb + 80, 16)]
            l256 = pari_v[pl.ds(pb + 96, 16)]
            ibase = l * 128
            acc0 = jnp.zeros((16,), jnp.float32)
            acc1 = jnp.zeros((16,), jnp.float32)
            for k in range(8):
                fpos = l256 + (k * 32) + p2
                f0 = plsc.load_gather(feat_buf, [fpos])
                f1 = plsc.load_gather(feat_buf, [fpos + 1])
                w = w_buf[pl.ds(ibase + k * 16, 16)]
                acc0 = acc0 + w * f0
                acc1 = acc1 + w * f1
            plsc.store_scatter(out_buf, [pout + l2v], acc0)
            plsc.store_scatter(out_buf, [pout + (l2v + 1)], acc1)
            return c

        lax.fori_loop(0, N_LEVELS, acc_body, 0)

        pltpu.async_copy(
            out_buf,
            out_hbm.at[pl.ds((base + g * _G) * _OUT_W, _G * _OUT_W)],
            outsem).wait()
        return carry

    lax.fori_loop(0, _NGROUPS, group_body, 0)


@jax.jit
def _run(points, anchors, feat_pool, bias_pool, res, pari):
    mesh = plsc.VectorSubcoreMesh(core_axis_name="c", subcore_axis_name="s")
    f = functools.partial(
        pl.kernel,
        out_type=jax.ShapeDtypeStruct((N_POINTS * _OUT_W,), jnp.float32),
        mesh=mesh,
        scratch_types=[
            pltpu.VMEM((_CHUNK * 3,), jnp.float32),
            pltpu.VMEM((_CHUNK,), jnp.int32),
            pltpu.VMEM((N_VOLUMES * N_LEVELS * 3,), jnp.float32),
            pltpu.VMEM((N_LEVELS * 16,), jnp.float32),
            pltpu.VMEM((N_LEVELS * _NPAR_I * 16,), jnp.int32),
            pltpu.VMEM((N_LEVELS * 256,), jnp.int32),
            pltpu.VMEM((N_LEVELS * 128,), jnp.float32),
            pltpu.VMEM((N_LEVELS * 256,), jnp.float32),
            pltpu.VMEM((_G * _OUT_W,), jnp.float32),
            pltpu.SemaphoreType.DMA,
            pltpu.SemaphoreType.DMA,
        ],
        compiler_params=pltpu.CompilerParams(needs_layout_passes=False),
    )(_body)
    return f(points, anchors, feat_pool, bias_pool, res, pari)


def kernel(points, anchors, feat_pool, bias_pool):
    levels = jnp.arange(N_LEVELS, dtype=jnp.float32)
    res = jnp.exp2(RES_BASE_POW2
                   + (RES_FINE_POW2 - RES_BASE_POW2) * levels / (N_LEVELS - 1))
    res_rep = jnp.broadcast_to(res[:, None], (N_LEVELS, 16)).reshape(-1)
    pari = jnp.asarray(_PARAMS_I_NP)
    out = _run(points.reshape(-1), anchors, feat_pool.reshape(-1),
               bias_pool.reshape(-1), res_rep, pari)
    return out.reshape(N_POINTS, _OUT_W)


# speedup vs baseline: 1.0062x; 1.0000x over previous
import functools

import jax
import jax.numpy as jnp
import numpy as np
from jax import lax
from jax.experimental import pallas as pl
from jax.experimental.pallas import tpu as pltpu
from jax.experimental.pallas import tpu_sc as plsc

N_LEVELS = 16
N_CHANNELS = 2
LOG2_TABLE = 19
N_VOLUMES = 64
LOCAL_SIZE = 1 << LOG2_TABLE
POOL_SIZE = LOCAL_SIZE * N_LEVELS
RES_BASE_POW2 = 3.0
RES_FINE_POW2 = 10.0
N_POINTS = 131072

_rng = np.random.RandomState(123)
_PRIMES_NP = (
    _rng.randint(1 << 20, 1 << 30, size=(N_LEVELS, 3)).astype(np.uint32)
    * np.uint32(2)
    + np.uint32(1)
).astype(np.int64).astype(np.int32)

_NC = 2
_NS = 16
_NW = _NC * _NS
_CHUNK = N_POINTS // _NW
_G = 16
_NGROUPS = _CHUNK // _G
_MASK = LOCAL_SIZE - 1
_OUT_W = N_LEVELS * N_CHANNELS

_NPAR_I = 7
_PARAMS_I_NP = np.zeros((N_LEVELS, _NPAR_I, 16), np.int32)
for _l in range(N_LEVELS):
    _PARAMS_I_NP[_l, 0, :] = _PRIMES_NP[_l, 0]
    _PARAMS_I_NP[_l, 1, :] = _PRIMES_NP[_l, 1]
    _PARAMS_I_NP[_l, 2, :] = _PRIMES_NP[_l, 2]
    _PARAMS_I_NP[_l, 3, :] = _l * (2 * LOCAL_SIZE)
    _PARAMS_I_NP[_l, 4, :] = 3 * _l
    _PARAMS_I_NP[_l, 5, :] = 2 * _l
    _PARAMS_I_NP[_l, 6, :] = 256 * _l
_PARAMS_I_NP = _PARAMS_I_NP.reshape(-1)


def _body(points_hbm, anchors_hbm, feat_hbm, bias_hbm, res_hbm, pari_hbm,
          out_hbm,
          pts_v, anch_v, bias_v, res_v, pari_v,
          idx_buf, w_buf, feat_buf, out_buf, sem, outsem):
    wid = lax.axis_index("s") * _NC + lax.axis_index("c")
    base = wid * _CHUNK

    pltpu.sync_copy(points_hbm.at[pl.ds(base * 3, _CHUNK * 3)], pts_v)
    pltpu.sync_copy(anchors_hbm.at[pl.ds(base, _CHUNK)], anch_v)
    pltpu.sync_copy(bias_hbm, bias_v)
    pltpu.sync_copy(res_hbm, res_v)
    pltpu.sync_copy(pari_hbm, pari_v)

    p_iota = lax.iota(jnp.int32, 16)
    p3 = p_iota * 3
    p2 = p_iota * 2
    pout = p_iota * _OUT_W
    zero_i = jnp.zeros((16,), jnp.int32)

    def group_body(g, carry):
        prow = g * (_G * 3) + p3
        x = plsc.load_gather(pts_v, [prow])
        y = plsc.load_gather(pts_v, [prow + 1])
        z = plsc.load_gather(pts_v, [prow + 2])
        anch = anch_v[pl.ds(g * _G, _G)]
        brow = anch * (N_LEVELS * 3)

        def level_body(l, c):
            pb = l * (_NPAR_I * 16)
            pxv = pari_v[pl.ds(pb, 16)]
            pyv = pari_v[pl.ds(pb + 16, 16)]
            pzv = pari_v[pl.ds(pb + 32, 16)]
            loff2 = pari_v[pl.ds(pb + 48, 16)]
            l3v = pari_v[pl.ds(pb + 64, 16)]
            l256 = pari_v[pl.ds(pb + 96, 16)]
            res = res_v[pl.ds(l * 16, 16)]
            bx = plsc.load_gather(bias_v, [brow + l3v])
            by = plsc.load_gather(bias_v, [brow + (l3v + 1)])
            bz = plsc.load_gather(bias_v, [brow + (l3v + 2)])
            ptx = x * res + bx
            pty = y * res + by
            ptz = z * res + bz
            ix = ptx.astype(jnp.int32)
            iy = pty.astype(jnp.int32)
            iz = ptz.astype(jnp.int32)
            fx = ptx - ix.astype(jnp.float32)
            fy = pty - iy.astype(jnp.float32)
            fz = ptz - iz.astype(jnp.float32)
            hx0 = ix * pxv
            hx1 = hx0 + pxv
            hy0 = iy * pyv
            hy1 = hy0 + pyv
            hz0 = iz * pzv
            hz1 = hz0 + pzv
            hx = (hx0, hx1)
            hy = (hy0, hy1)
            hz = (hz0, hz1)
            wx = (1.0 - fx, fx)
            wy = (1.0 - fy, fy)
            wz = (1.0 - fz, fz)
            ibase = l * 128
            ibase2 = l * 256
            for k in range(8):
                bxk, byk, bzk = (k >> 2) & 1, (k >> 1) & 1, k & 1
                hm = (hx[bxk] ^ hy[byk] ^ hz[bzk]) & _MASK
                h2 = hm + hm + loff2
                posv = l256 + (k * 32) + p2
                plsc.store_scatter(idx_buf, [posv], h2)
                plsc.store_scatter(idx_buf, [posv + 1], h2 + 1)
                w_buf[pl.ds(ibase + k * 16, 16)] = wx[bxk] * wy[byk] * wz[bzk]
            for j in range(2):
                sl = pl.ds(ibase2 + j * 128, 128)
                pltpu.async_copy(feat_hbm.at[idx_buf.at[sl]],
                                 feat_buf.at[sl], sem)
            return c

        lax.fori_loop(0, N_LEVELS, level_body, 0)

        def drain_body(l, c):
            for j in range(2):
                sl = pl.ds(l * 256 + j * 128, 128)
                pltpu.make_async_copy(feat_hbm.at[idx_buf.at[sl]],
                                      feat_buf.at[sl], sem).wait()
            return c

        lax.fori_loop(0, N_LEVELS, drain_body, 0)

        def acc_body(l, c):
            pb = l * (_NPAR_I * 16)
            l2v = pari_v[pl.ds(pb + 80, 16)]
            l256 = pari_v[pl.ds(pb + 96, 16)]
            ibase = l * 128
            acc0 = jnp.zeros((16,), jnp.float32)
            acc1 = jnp.zeros((16,), jnp.float32)
            for k in range(8):
                fpos = l256 + (k * 32) + p2
                f0 = plsc.load_gather(feat_buf, [fpos])
                f1 = plsc.load_gather(feat_buf, [fpos + 1])
                w = w_buf[pl.ds(ibase + k * 16, 16)]
                acc0 = acc0 + w * f0
                acc1 = acc1 + w * f1
            plsc.store_scatter(out_buf, [pout + l2v], acc0)
            plsc.store_scatter(out_buf, [pout + (l2v + 1)], acc1)
            return c

        lax.fori_loop(0, N_LEVELS, acc_body, 0)

        pltpu.async_copy(
            out_buf,
            out_hbm.at[pl.ds((base + g * _G) * _OUT_W, _G * _OUT_W)],
            outsem).wait()
        return carry

    lax.fori_loop(0, _NGROUPS, group_body, 0)


@jax.jit
def _run(points, anchors, feat_pool, bias_pool, res, pari):
    mesh = plsc.VectorSubcoreMesh(core_axis_name="c", subcore_axis_name="s")
    f = functools.partial(
        pl.kernel,
        out_type=jax.ShapeDtypeStruct((N_POINTS * _OUT_W,), jnp.float32),
        mesh=mesh,
        scratch_types=[
            pltpu.VMEM((_CHUNK * 3,), jnp.float32),
            pltpu.VMEM((_CHUNK,), jnp.int32),
            pltpu.VMEM((N_VOLUMES * N_LEVELS * 3,), jnp.float32),
            pltpu.VMEM((N_LEVELS * 16,), jnp.float32),
            pltpu.VMEM((N_LEVELS * _NPAR_I * 16,), jnp.int32),
            pltpu.VMEM((N_LEVELS * 256,), jnp.int32),
            pltpu.VMEM((N_LEVELS * 128,), jnp.float32),
            pltpu.VMEM((N_LEVELS * 256,), jnp.float32),
            pltpu.VMEM((_G * _OUT_W,), jnp.float32),
            pltpu.SemaphoreType.DMA,
            pltpu.SemaphoreType.DMA,
        ],
        compiler_params=pltpu.CompilerParams(needs_layout_passes=False),
    )(_body)
    return f(points, anchors, feat_pool, bias_pool, res, pari)


def kernel(points, anchors, feat_pool, bias_pool):
    levels = jnp.arange(N_LEVELS, dtype=jnp.float32)
    res = jnp.exp2(RES_BASE_POW2
                   + (RES_FINE_POW2 - RES_BASE_POW2) * levels / (N_LEVELS - 1))
    res_rep = jnp.broadcast_to(res[:, None], (N_LEVELS, 16)).reshape(-1)
    pari = jnp.asarray(_PARAMS_I_NP)
    one = (anchors[0] * 0 + 1).astype(jnp.float32)
    out = _run(points.reshape(-1), anchors, feat_pool.reshape(-1) * one,
               bias_pool.reshape(-1), res_rep, pari)
    return out.reshape(N_POINTS, _OUT_W)


# speedup vs baseline: 7.0173x; 6.9738x over previous
import functools

import jax
import jax.numpy as jnp
import numpy as np
from jax import lax
from jax.experimental import pallas as pl
from jax.experimental.pallas import tpu as pltpu
from jax.experimental.pallas import tpu_sc as plsc

N_LEVELS = 16
N_CHANNELS = 2
LOG2_TABLE = 19
N_VOLUMES = 64
LOCAL_SIZE = 1 << LOG2_TABLE
POOL_SIZE = LOCAL_SIZE * N_LEVELS
RES_BASE_POW2 = 3.0
RES_FINE_POW2 = 10.0
N_POINTS = 131072

_rng = np.random.RandomState(123)
_PRIMES_NP = (
    _rng.randint(1 << 20, 1 << 30, size=(N_LEVELS, 3)).astype(np.uint32)
    * np.uint32(2)
    + np.uint32(1)
).astype(np.int64).astype(np.int32)

_NC = 2
_NS = 16
_NW = _NC * _NS
_CHUNK = N_POINTS // _NW
_G = 16
_NGROUPS = _CHUNK // _G
_MASK = LOCAL_SIZE - 1
_OUT_W = N_LEVELS * N_CHANNELS

_NPAR_I = 6
_PARAMS_I_NP = np.zeros((N_LEVELS, _NPAR_I, 16), np.int32)
for _l in range(N_LEVELS):
    _PARAMS_I_NP[_l, 0, :] = _PRIMES_NP[_l, 0]
    _PARAMS_I_NP[_l, 1, :] = _PRIMES_NP[_l, 1]
    _PARAMS_I_NP[_l, 2, :] = _PRIMES_NP[_l, 2]
    _PARAMS_I_NP[_l, 3, :] = _l * LOCAL_SIZE
    _PARAMS_I_NP[_l, 4, :] = 3 * _l
    _PARAMS_I_NP[_l, 5, :] = 2 * _l
_PARAMS_I_NP = _PARAMS_I_NP.reshape(-1)


def _body(points_hbm, anchors_hbm, ch0_hbm, ch1_hbm, bias_hbm, res_hbm,
          pari_hbm,
          out_hbm,
          pts_v, anch_v, bias_v, res_v, pari_v,
          idx_buf, w_buf, f0_buf, f1_buf, out_buf, sem, outsem):
    wid = lax.axis_index("s") * _NC + lax.axis_index("c")
    base = wid * _CHUNK

    pltpu.sync_copy(points_hbm.at[pl.ds(base * 3, _CHUNK * 3)], pts_v)
    pltpu.sync_copy(anchors_hbm.at[pl.ds(base, _CHUNK)], anch_v)
    pltpu.sync_copy(bias_hbm, bias_v)
    pltpu.sync_copy(res_hbm, res_v)
    pltpu.sync_copy(pari_hbm, pari_v)

    p_iota = lax.iota(jnp.int32, 16)
    p3 = p_iota * 3
    pout = p_iota * _OUT_W

    def group_body(g, carry):
        prow = g * (_G * 3) + p3
        x = plsc.load_gather(pts_v, [prow])
        y = plsc.load_gather(pts_v, [prow + 1])
        z = plsc.load_gather(pts_v, [prow + 2])
        anch = anch_v[pl.ds(g * _G, _G)]
        brow = anch * (N_LEVELS * 3)

        def level_body(l, c):
            pb = l * (_NPAR_I * 16)
            pxv = pari_v[pl.ds(pb, 16)]
            pyv = pari_v[pl.ds(pb + 16, 16)]
            pzv = pari_v[pl.ds(pb + 32, 16)]
            loff = pari_v[pl.ds(pb + 48, 16)]
            l3v = pari_v[pl.ds(pb + 64, 16)]
            res = res_v[pl.ds(l * 16, 16)]
            bx = plsc.load_gather(bias_v, [brow + l3v])
            by = plsc.load_gather(bias_v, [brow + (l3v + 1)])
            bz = plsc.load_gather(bias_v, [brow + (l3v + 2)])
            ptx = x * res + bx
            pty = y * res + by
            ptz = z * res + bz
            ix = ptx.astype(jnp.int32)
            iy = pty.astype(jnp.int32)
            iz = ptz.astype(jnp.int32)
            fx = ptx - ix.astype(jnp.float32)
            fy = pty - iy.astype(jnp.float32)
            fz = ptz - iz.astype(jnp.float32)
            hx0 = ix * pxv
            hx1 = hx0 + pxv
            hy0 = iy * pyv
            hy1 = hy0 + pyv
            hz0 = iz * pzv
            hz1 = hz0 + pzv
            hx = (hx0, hx1)
            hy = (hy0, hy1)
            hz = (hz0, hz1)
            wx = (1.0 - fx, fx)
            wy = (1.0 - fy, fy)
            wz = (1.0 - fz, fz)
            ibase = l * 128
            for k in range(8):
                bxk, byk, bzk = (k >> 2) & 1, (k >> 1) & 1, k & 1
                hm = (hx[bxk] ^ hy[byk] ^ hz[bzk]) & _MASK
                idx_buf[pl.ds(ibase + k * 16, 16)] = hm + loff
                w_buf[pl.ds(ibase + k * 16, 16)] = wx[bxk] * wy[byk] * wz[bzk]
            sl = pl.ds(ibase, 128)
            pltpu.async_copy(ch0_hbm.at[idx_buf.at[sl]], f0_buf.at[sl], sem)
            pltpu.async_copy(ch1_hbm.at[idx_buf.at[sl]], f1_buf.at[sl], sem)
            return c

        lax.fori_loop(0, N_LEVELS, level_body, 0)

        def drain_body(l, c):
            sl = pl.ds(l * 128, 128)
            pltpu.make_async_copy(ch0_hbm.at[idx_buf.at[sl]],
                                  f0_buf.at[sl], sem).wait()
            pltpu.make_async_copy(ch1_hbm.at[idx_buf.at[sl]],
                                  f1_buf.at[sl], sem).wait()
            return c

        lax.fori_loop(0, N_LEVELS, drain_body, 0)

        def acc_body(l, c):
            pb = l * (_NPAR_I * 16)
            l2v = pari_v[pl.ds(pb + 80, 16)]
            ibase = l * 128
            acc0 = jnp.zeros((16,), jnp.float32)
            acc1 = jnp.zeros((16,), jnp.float32)
            for k in range(8):
                sl = pl.ds(ibase + k * 16, 16)
                w = w_buf[sl]
                acc0 = acc0 + w * f0_buf[sl]
                acc1 = acc1 + w * f1_buf[sl]
            plsc.store_scatter(out_buf, [pout + l2v], acc0)
            plsc.store_scatter(out_buf, [pout + (l2v + 1)], acc1)
            return c

        lax.fori_loop(0, N_LEVELS, acc_body, 0)

        pltpu.async_copy(
            out_buf,
            out_hbm.at[pl.ds((base + g * _G) * _OUT_W, _G * _OUT_W)],
            outsem).wait()
        return carry

    lax.fori_loop(0, _NGROUPS, group_body, 0)


@jax.jit
def _run(points, anchors, ch0, ch1, bias_pool, res, pari):
    mesh = plsc.VectorSubcoreMesh(core_axis_name="c", subcore_axis_name="s")
    f = functools.partial(
        pl.kernel,
        out_type=jax.ShapeDtypeStruct((N_POINTS * _OUT_W,), jnp.float32),
        mesh=mesh,
        scratch_types=[
            pltpu.VMEM((_CHUNK * 3,), jnp.float32),
            pltpu.VMEM((_CHUNK,), jnp.int32),
            pltpu.VMEM((N_VOLUMES * N_LEVELS * 3,), jnp.float32),
            pltpu.VMEM((N_LEVELS * 16,), jnp.float32),
            pltpu.VMEM((N_LEVELS * _NPAR_I * 16,), jnp.int32),
            pltpu.VMEM((N_LEVELS * 128,), jnp.int32),
            pltpu.VMEM((N_LEVELS * 128,), jnp.float32),
            pltpu.VMEM((N_LEVELS * 128,), jnp.float32),
            pltpu.VMEM((N_LEVELS * 128,), jnp.float32),
            pltpu.VMEM((_G * _OUT_W,), jnp.float32),
            pltpu.SemaphoreType.DMA,
            pltpu.SemaphoreType.DMA,
        ],
        compiler_params=pltpu.CompilerParams(needs_layout_passes=False),
    )(_body)
    return f(points, anchors, ch0, ch1, bias_pool, res, pari)


def kernel(points, anchors, feat_pool, bias_pool):
    levels = jnp.arange(N_LEVELS, dtype=jnp.float32)
    res = jnp.exp2(RES_BASE_POW2
                   + (RES_FINE_POW2 - RES_BASE_POW2) * levels / (N_LEVELS - 1))
    res_rep = jnp.broadcast_to(res[:, None], (N_LEVELS, 16)).reshape(-1)
    pari = jnp.asarray(_PARAMS_I_NP)
    ch0 = feat_pool[:, 0]
    ch1 = feat_pool[:, 1]
    out = _run(points.reshape(-1), anchors, ch0, ch1,
               bias_pool.reshape(-1), res_rep, pari)
    return out.reshape(N_POINTS, _OUT_W)


# speedup vs baseline: 7.0924x; 1.0107x over previous
import functools

import jax
import jax.numpy as jnp
import numpy as np
from jax import lax
from jax.experimental import pallas as pl
from jax.experimental.pallas import tpu as pltpu
from jax.experimental.pallas import tpu_sc as plsc

N_LEVELS = 16
N_CHANNELS = 2
LOG2_TABLE = 19
N_VOLUMES = 64
LOCAL_SIZE = 1 << LOG2_TABLE
POOL_SIZE = LOCAL_SIZE * N_LEVELS
RES_BASE_POW2 = 3.0
RES_FINE_POW2 = 10.0
N_POINTS = 131072

_rng = np.random.RandomState(123)
_PRIMES_NP = (
    _rng.randint(1 << 20, 1 << 30, size=(N_LEVELS, 3)).astype(np.uint32)
    * np.uint32(2)
    + np.uint32(1)
).astype(np.int64).astype(np.int32)

_NC = 2
_NS = 16
_NW = _NC * _NS
_CHUNK = N_POINTS // _NW
_G = 16
_NGROUPS = _CHUNK // _G
_MASK = LOCAL_SIZE - 1
_OUT_W = N_LEVELS * N_CHANNELS

_NPAR_I = 6
_PARAMS_I_NP = np.zeros((N_LEVELS, _NPAR_I, 16), np.int32)
for _l in range(N_LEVELS):
    _PARAMS_I_NP[_l, 0, :] = _PRIMES_NP[_l, 0]
    _PARAMS_I_NP[_l, 1, :] = _PRIMES_NP[_l, 1]
    _PARAMS_I_NP[_l, 2, :] = _PRIMES_NP[_l, 2]
    _PARAMS_I_NP[_l, 3, :] = _l * LOCAL_SIZE
    _PARAMS_I_NP[_l, 4, :] = 3 * _l
    _PARAMS_I_NP[_l, 5, :] = 2 * _l
_PARAMS_I_NP = _PARAMS_I_NP.reshape(-1)


def _body(points_hbm, anchors_hbm, ch0_hbm, ch1_hbm, bias_hbm, res_hbm,
          pari_hbm,
          out_hbm,
          pts_v, anch_v, bias_v, res_v, pari_v,
          idx_buf, w_buf, f0_buf, f1_buf, out_buf, sem, outsem):
    wid = lax.axis_index("s") * _NC + lax.axis_index("c")
    base = wid * _CHUNK

    pltpu.sync_copy(points_hbm.at[pl.ds(base * 3, _CHUNK * 3)], pts_v)
    pltpu.sync_copy(anchors_hbm.at[pl.ds(base, _CHUNK)], anch_v)
    pltpu.sync_copy(bias_hbm, bias_v)
    pltpu.sync_copy(res_hbm, res_v)
    pltpu.sync_copy(pari_hbm, pari_v)

    p_iota = lax.iota(jnp.int32, 16)
    p3 = p_iota * 3
    pout = p_iota * _OUT_W

    def group_body(g, carry):
        prow = g * (_G * 3) + p3
        x = plsc.load_gather(pts_v, [prow])
        y = plsc.load_gather(pts_v, [prow + 1])
        z = plsc.load_gather(pts_v, [prow + 2])
        anch = anch_v[pl.ds(g * _G, _G)]
        brow = anch * (N_LEVELS * 3)

        def level_body(l, c):
            pb = l * (_NPAR_I * 16)
            pxv = pari_v[pl.ds(pb, 16)]
            pyv = pari_v[pl.ds(pb + 16, 16)]
            pzv = pari_v[pl.ds(pb + 32, 16)]
            loff = pari_v[pl.ds(pb + 48, 16)]
            l3v = pari_v[pl.ds(pb + 64, 16)]
            res = res_v[pl.ds(l * 16, 16)]
            bx = plsc.load_gather(bias_v, [brow + l3v])
            by = plsc.load_gather(bias_v, [brow + (l3v + 1)])
            bz = plsc.load_gather(bias_v, [brow + (l3v + 2)])
            ptx = x * res + bx
            pty = y * res + by
            ptz = z * res + bz
            ix = ptx.astype(jnp.int32)
            iy = pty.astype(jnp.int32)
            iz = ptz.astype(jnp.int32)
            fx = ptx - ix.astype(jnp.float32)
            fy = pty - iy.astype(jnp.float32)
            fz = ptz - iz.astype(jnp.float32)
            hx0 = ix * pxv
            hx1 = hx0 + pxv
            hy0 = iy * pyv
            hy1 = hy0 + pyv
            hz0 = iz * pzv
            hz1 = hz0 + pzv
            hx = (hx0, hx1)
            hy = (hy0, hy1)
            hz = (hz0, hz1)
            wx = (1.0 - fx, fx)
            wy = (1.0 - fy, fy)
            wz = (1.0 - fz, fz)
            ibase = l * 128
            for k in range(8):
                bxk, byk, bzk = (k >> 2) & 1, (k >> 1) & 1, k & 1
                hm = (hx[bxk] ^ hy[byk] ^ hz[bzk]) & _MASK
                idx_buf[pl.ds(ibase + k * 16, 16)] = hm + loff
                w_buf[pl.ds(ibase + k * 16, 16)] = wx[bxk] * wy[byk] * wz[bzk]
            sl = pl.ds(ibase, 128)
            pltpu.async_copy(ch0_hbm.at[idx_buf.at[sl]], f0_buf.at[sl], sem)
            pltpu.async_copy(ch1_hbm.at[idx_buf.at[sl]], f1_buf.at[sl], sem)
            return c

        lax.fori_loop(0, N_LEVELS, level_body, 0, unroll=2)

        def drain_body(l, c):
            sl = pl.ds(l * 128, 128)
            pltpu.make_async_copy(ch0_hbm.at[idx_buf.at[sl]],
                                  f0_buf.at[sl], sem).wait()
            pltpu.make_async_copy(ch1_hbm.at[idx_buf.at[sl]],
                                  f1_buf.at[sl], sem).wait()
            return c

        lax.fori_loop(0, N_LEVELS, drain_body, 0)

        @pl.when(g > 0)
        def _wait_prev():
            gp = g - 1
            pltpu.make_async_copy(
                out_buf.at[pl.ds((gp % 2) * (_G * _OUT_W), _G * _OUT_W)],
                out_hbm.at[pl.ds((base + gp * _G) * _OUT_W, _G * _OUT_W)],
                outsem).wait()

        obase = (g % 2) * (_G * _OUT_W)

        def acc_body(l, c):
            pb = l * (_NPAR_I * 16)
            l2v = pari_v[pl.ds(pb + 80, 16)]
            ibase = l * 128
            acc0 = jnp.zeros((16,), jnp.float32)
            acc1 = jnp.zeros((16,), jnp.float32)
            for k in range(8):
                sl = pl.ds(ibase + k * 16, 16)
                w = w_buf[sl]
                acc0 = acc0 + w * f0_buf[sl]
                acc1 = acc1 + w * f1_buf[sl]
            plsc.store_scatter(out_buf, [obase + (pout + l2v)], acc0)
            plsc.store_scatter(out_buf, [obase + (pout + (l2v + 1))], acc1)
            return c

        lax.fori_loop(0, N_LEVELS, acc_body, 0, unroll=2)

        pltpu.async_copy(
            out_buf.at[pl.ds(obase, _G * _OUT_W)],
            out_hbm.at[pl.ds((base + g * _G) * _OUT_W, _G * _OUT_W)],
            outsem)
        return carry

    lax.fori_loop(0, _NGROUPS, group_body, 0)
    gl = _NGROUPS - 1
    pltpu.make_async_copy(
        out_buf.at[pl.ds((gl % 2) * (_G * _OUT_W), _G * _OUT_W)],
        out_hbm.at[pl.ds((base + gl * _G) * _OUT_W, _G * _OUT_W)],
        outsem).wait()


@jax.jit
def _run(points, anchors, ch0, ch1, bias_pool, res, pari):
    mesh = plsc.VectorSubcoreMesh(core_axis_name="c", subcore_axis_name="s")
    f = functools.partial(
        pl.kernel,
        out_type=jax.ShapeDtypeStruct((N_POINTS * _OUT_W,), jnp.float32),
        mesh=mesh,
        scratch_types=[
            pltpu.VMEM((_CHUNK * 3,), jnp.float32),
            pltpu.VMEM((_CHUNK,), jnp.int32),
            pltpu.VMEM((N_VOLUMES * N_LEVELS * 3,), jnp.float32),
            pltpu.VMEM((N_LEVELS * 16,), jnp.float32),
            pltpu.VMEM((N_LEVELS * _NPAR_I * 16,), jnp.int32),
            pltpu.VMEM((N_LEVELS * 128,), jnp.int32),
            pltpu.VMEM((N_LEVELS * 128,), jnp.float32),
            pltpu.VMEM((N_LEVELS * 128,), jnp.float32),
            pltpu.VMEM((N_LEVELS * 128,), jnp.float32),
            pltpu.VMEM((2 * _G * _OUT_W,), jnp.float32),
            pltpu.SemaphoreType.DMA,
            pltpu.SemaphoreType.DMA,
        ],
        compiler_params=pltpu.CompilerParams(needs_layout_passes=False),
    )(_body)
    return f(points, anchors, ch0, ch1, bias_pool, res, pari)


def kernel(points, anchors, feat_pool, bias_pool):
    levels = jnp.arange(N_LEVELS, dtype=jnp.float32)
    res = jnp.exp2(RES_BASE_POW2
                   + (RES_FINE_POW2 - RES_BASE_POW2) * levels / (N_LEVELS - 1))
    res_rep = jnp.broadcast_to(res[:, None], (N_LEVELS, 16)).reshape(-1)
    pari = jnp.asarray(_PARAMS_I_NP)
    ch0 = feat_pool[:, 0]
    ch1 = feat_pool[:, 1]
    out = _run(points.reshape(-1), anchors, ch0, ch1,
               bias_pool.reshape(-1), res_rep, pari)
    return out.reshape(N_POINTS, _OUT_W)


# speedup vs baseline: 8.5894x; 1.2111x over previous
import functools

import jax
import jax.numpy as jnp
import numpy as np
from jax import lax
from jax.experimental import pallas as pl
from jax.experimental.pallas import tpu as pltpu
from jax.experimental.pallas import tpu_sc as plsc

N_LEVELS = 16
N_CHANNELS = 2
LOG2_TABLE = 19
N_VOLUMES = 64
LOCAL_SIZE = 1 << LOG2_TABLE
POOL_SIZE = LOCAL_SIZE * N_LEVELS
RES_BASE_POW2 = 3.0
RES_FINE_POW2 = 10.0
N_POINTS = 131072

_rng = np.random.RandomState(123)
_PRIMES_NP = (
    _rng.randint(1 << 20, 1 << 30, size=(N_LEVELS, 3)).astype(np.uint32)
    * np.uint32(2)
    + np.uint32(1)
).astype(np.int64).astype(np.int32)

_NC = 2
_NS = 16
_NW = _NC * _NS
_CHUNK = N_POINTS // _NW
_G = 16
_NGROUPS = _CHUNK // _G
_MASK = LOCAL_SIZE - 1
_OUT_W = N_LEVELS * N_CHANNELS

_NPAR_I = 6
_PARAMS_I_NP = np.zeros((N_LEVELS, _NPAR_I, 16), np.int32)
for _l in range(N_LEVELS):
    _PARAMS_I_NP[_l, 0, :] = _PRIMES_NP[_l, 0]
    _PARAMS_I_NP[_l, 1, :] = _PRIMES_NP[_l, 1]
    _PARAMS_I_NP[_l, 2, :] = _PRIMES_NP[_l, 2]
    _PARAMS_I_NP[_l, 3, :] = _l * LOCAL_SIZE
    _PARAMS_I_NP[_l, 4, :] = 3 * _l
    _PARAMS_I_NP[_l, 5, :] = 2 * _l
_PARAMS_I_NP = _PARAMS_I_NP.reshape(-1)


def _body(points_hbm, anchors_hbm, ch0_hbm, ch1_hbm, bias_hbm, res_hbm,
          pari_hbm,
          out_hbm,
          pts_v, anch_v, bias_v, res_v, pari_v,
          idx_buf, w_buf, f0_buf, f1_buf, out_buf, sem_a, sem_b, outsem):
    wid = lax.axis_index("s") * _NC + lax.axis_index("c")
    base = wid * _CHUNK

    pltpu.sync_copy(points_hbm.at[pl.ds(base * 3, _CHUNK * 3)], pts_v)
    pltpu.sync_copy(anchors_hbm.at[pl.ds(base, _CHUNK)], anch_v)
    pltpu.sync_copy(bias_hbm, bias_v)
    pltpu.sync_copy(res_hbm, res_v)
    pltpu.sync_copy(pari_hbm, pari_v)

    p_iota = lax.iota(jnp.int32, 16)
    p3 = p_iota * 3
    pout = p_iota * _OUT_W

    _B = N_LEVELS * 128

    def fire(g, ob, sem):
        prow = g * (_G * 3) + p3
        x = plsc.load_gather(pts_v, [prow])
        y = plsc.load_gather(pts_v, [prow + 1])
        z = plsc.load_gather(pts_v, [prow + 2])
        anch = anch_v[pl.ds(g * _G, _G)]
        brow = anch * (N_LEVELS * 3)

        def level_body(l, c):
            pb = l * (_NPAR_I * 16)
            pxv = pari_v[pl.ds(pb, 16)]
            pyv = pari_v[pl.ds(pb + 16, 16)]
            pzv = pari_v[pl.ds(pb + 32, 16)]
            loff = pari_v[pl.ds(pb + 48, 16)]
            l3v = pari_v[pl.ds(pb + 64, 16)]
            res = res_v[pl.ds(l * 16, 16)]
            bx = plsc.load_gather(bias_v, [brow + l3v])
            by = plsc.load_gather(bias_v, [brow + (l3v + 1)])
            bz = plsc.load_gather(bias_v, [brow + (l3v + 2)])
            ptx = x * res + bx
            pty = y * res + by
            ptz = z * res + bz
            ix = ptx.astype(jnp.int32)
            iy = pty.astype(jnp.int32)
            iz = ptz.astype(jnp.int32)
            fx = ptx - ix.astype(jnp.float32)
            fy = pty - iy.astype(jnp.float32)
            fz = ptz - iz.astype(jnp.float32)
            hx0 = ix * pxv
            hx1 = hx0 + pxv
            hy0 = iy * pyv
            hy1 = hy0 + pyv
            hz0 = iz * pzv
            hz1 = hz0 + pzv
            hx = (hx0, hx1)
            hy = (hy0, hy1)
            hz = (hz0, hz1)
            wx = (1.0 - fx, fx)
            wy = (1.0 - fy, fy)
            wz = (1.0 - fz, fz)
            ibase = ob + l * 128
            for k in range(8):
                bxk, byk, bzk = (k >> 2) & 1, (k >> 1) & 1, k & 1
                hm = (hx[bxk] ^ hy[byk] ^ hz[bzk]) & _MASK
                idx_buf[pl.ds(ibase + k * 16, 16)] = hm + loff
                w_buf[pl.ds(ibase + k * 16, 16)] = wx[bxk] * wy[byk] * wz[bzk]
            sl = pl.ds(ibase, 128)
            pltpu.async_copy(ch0_hbm.at[idx_buf.at[sl]], f0_buf.at[sl], sem)
            pltpu.async_copy(ch1_hbm.at[idx_buf.at[sl]], f1_buf.at[sl], sem)
            return c

        lax.fori_loop(0, N_LEVELS, level_body, 0, unroll=2)

    def drain(ob, sem):
        def drain_body(l, c):
            sl = pl.ds(ob + l * 128, 128)
            pltpu.make_async_copy(ch0_hbm.at[idx_buf.at[sl]],
                                  f0_buf.at[sl], sem).wait()
            pltpu.make_async_copy(ch1_hbm.at[idx_buf.at[sl]],
                                  f1_buf.at[sl], sem).wait()
            return c

        lax.fori_loop(0, N_LEVELS, drain_body, 0)

    def accout(g, ob):
        @pl.when(g > 0)
        def _wait_prev():
            gp = g - 1
            pltpu.make_async_copy(
                out_buf.at[pl.ds((gp % 2) * (_G * _OUT_W), _G * _OUT_W)],
                out_hbm.at[pl.ds((base + gp * _G) * _OUT_W, _G * _OUT_W)],
                outsem).wait()

        obase = (g % 2) * (_G * _OUT_W)

        def acc_body(l, c):
            pb = l * (_NPAR_I * 16)
            l2v = pari_v[pl.ds(pb + 80, 16)]
            ibase = ob + l * 128
            acc0 = jnp.zeros((16,), jnp.float32)
            acc1 = jnp.zeros((16,), jnp.float32)
            for k in range(8):
                sl = pl.ds(ibase + k * 16, 16)
                w = w_buf[sl]
                acc0 = acc0 + w * f0_buf[sl]
                acc1 = acc1 + w * f1_buf[sl]
            plsc.store_scatter(out_buf, [obase + (pout + l2v)], acc0)
            plsc.store_scatter(out_buf, [obase + (pout + (l2v + 1))], acc1)
            return c

        lax.fori_loop(0, N_LEVELS, acc_body, 0, unroll=2)

        pltpu.async_copy(
            out_buf.at[pl.ds(obase, _G * _OUT_W)],
            out_hbm.at[pl.ds((base + g * _G) * _OUT_W, _G * _OUT_W)],
            outsem)

    def pair_body(i, carry):
        j0 = 2 * i
        j1 = j0 + 1
        fire(j0, 0, sem_a)

        @pl.when(i > 0)
        def _finish_prev_odd():
            drain(_B, sem_b)
            accout(j0 - 1, _B)

        fire(j1, _B, sem_b)
        drain(0, sem_a)
        accout(j0, 0)
        return carry

    lax.fori_loop(0, _NGROUPS // 2, pair_body, 0)
    jl = _NGROUPS - 1
    drain(_B, sem_b)
    accout(jl, _B)
    pltpu.make_async_copy(
        out_buf.at[pl.ds((jl % 2) * (_G * _OUT_W), _G * _OUT_W)],
        out_hbm.at[pl.ds((base + jl * _G) * _OUT_W, _G * _OUT_W)],
        outsem).wait()


@jax.jit
def _run(points, anchors, ch0, ch1, bias_pool, res, pari):
    mesh = plsc.VectorSubcoreMesh(core_axis_name="c", subcore_axis_name="s")
    f = functools.partial(
        pl.kernel,
        out_type=jax.ShapeDtypeStruct((N_POINTS * _OUT_W,), jnp.float32),
        mesh=mesh,
        scratch_types=[
            pltpu.VMEM((_CHUNK * 3,), jnp.float32),
            pltpu.VMEM((_CHUNK,), jnp.int32),
            pltpu.VMEM((N_VOLUMES * N_LEVELS * 3,), jnp.float32),
            pltpu.VMEM((N_LEVELS * 16,), jnp.float32),
            pltpu.VMEM((N_LEVELS * _NPAR_I * 16,), jnp.int32),
            pltpu.VMEM((2 * N_LEVELS * 128,), jnp.int32),
            pltpu.VMEM((2 * N_LEVELS * 128,), jnp.float32),
            pltpu.VMEM((2 * N_LEVELS * 128,), jnp.float32),
            pltpu.VMEM((2 * N_LEVELS * 128,), jnp.float32),
            pltpu.VMEM((2 * _G * _OUT_W,), jnp.float32),
            pltpu.SemaphoreType.DMA,
            pltpu.SemaphoreType.DMA,
            pltpu.SemaphoreType.DMA,
        ],
        compiler_params=pltpu.CompilerParams(needs_layout_passes=False),
    )(_body)
    return f(points, anchors, ch0, ch1, bias_pool, res, pari)


def kernel(points, anchors, feat_pool, bias_pool):
    levels = jnp.arange(N_LEVELS, dtype=jnp.float32)
    res = jnp.exp2(RES_BASE_POW2
                   + (RES_FINE_POW2 - RES_BASE_POW2) * levels / (N_LEVELS - 1))
    res_rep = jnp.broadcast_to(res[:, None], (N_LEVELS, 16)).reshape(-1)
    pari = jnp.asarray(_PARAMS_I_NP)
    ch0 = feat_pool[:, 0]
    ch1 = feat_pool[:, 1]
    out = _run(points.reshape(-1), anchors, ch0, ch1,
               bias_pool.reshape(-1), res_rep, pari)
    return out.reshape(N_POINTS, _OUT_W)
